# baseline (speedup 1.0000x reference)
import jax
import jax.numpy as jnp
from jax.experimental import pallas as pl

B, C, D, H, W = 4, 128, 2, 200, 176
S = D * H * W


def kernel(features, batch_idx, z_idx, y_idx, x_idx):
    N = features.shape[0]
    q = ((batch_idx * D + z_idx) * H + y_idx) * W + x_idx
    wid = jnp.zeros((B * S,), jnp.int32).at[q].max(jnp.arange(N, dtype=jnp.int32) + 1)
    rows = features[jnp.maximum(wid - 1, 0)]
    rows = jnp.where((wid > 0)[:, None], rows, 0.0)
    out = rows.reshape(B, D, H, W, C).transpose(0, 4, 1, 2, 3)
    return out


# speedup vs baseline: 1.1094x; 1.1094x over previous
import functools

import jax
import jax.numpy as jnp
from jax import lax
from jax.experimental import pallas as pl
from jax.experimental.pallas import tpu as pltpu
from jax.experimental.pallas import tpu_sc as plsc

B, C, D, H, W = 4, 128, 2, 200, 176
S = D * H * W
Q = B * S
N = 40000
NT = 32
TQ = Q // NT
KW = 176
NWIN = TQ // KW
SCAN = 4000
NCH = N // SCAN
VPC = SCAN // 16
GCAP = 208
IMAX = 0x7FFFFFFF


def _body(feat, bidx, zidx, yidx, xidx, out,
          wid, bbuf, zbuf, ybuf, xbuf, sc17,
          jbufA, jbufB, idxbuf, rows, outTA, outTB,
          gsem, osemA, osemB):
    t = lax.axis_index("c") * 16 + lax.axis_index("s")
    q0 = t * TQ
    bt = t // 8
    sbase = (t % 8) * TQ
    iota = lax.iota(jnp.int32, 16)
    z16f = jnp.zeros((16,), jnp.float32)
    z16i = jnp.zeros((16,), jnp.int32)

    def zwid(k, _):
        wid[pl.ds(k * 16, 16)] = z16i
        return 0
    lax.fori_loop(0, TQ // 16, zwid, 0)
    sc17[pl.ds(16, 16)] = jnp.full((16,), -1, jnp.int32)

    def zot(k, _):
        c = k // (KW // 16)
        o = (k % (KW // 16)) * 16
        outTA[c, pl.ds(o, 16)] = z16f
        outTB[c, pl.ds(o, 16)] = z16f
        return 0
    lax.fori_loop(0, C * (KW // 16), zot, 0)

    def chunk_body(ch, _):
        off = ch * SCAN
        pltpu.sync_copy(bidx.at[pl.ds(off, SCAN)], bbuf)
        pltpu.sync_copy(zidx.at[pl.ds(off, SCAN)], zbuf)
        pltpu.sync_copy(yidx.at[pl.ds(off, SCAN)], ybuf)
        pltpu.sync_copy(xidx.at[pl.ds(off, SCAN)], xbuf)

        def vec_body(k, _):
            n_vec = (off + k * 16) + iota
            bv = bbuf[pl.ds(k * 16, 16)]
            zv = zbuf[pl.ds(k * 16, 16)]
            yv = ybuf[pl.ds(k * 16, 16)]
            xv = xbuf[pl.ds(k * 16, 16)]
            ql = ((bv * D + zv) * H + yv) * W + xv - q0
            inr = (ql >= 0) & (ql < TQ)
            key = jnp.where(inr, (ql << 16) | n_vec, IMAX)
            sk, sv = plsc.sort_key_val(key, n_vec + 1)
            qs = sk >> 16
            sc17[pl.ds(0, 16)] = sk
            nk = plsc.load_gather(sc17, [iota + 1])
            keep = (qs < TQ) & (qs != (nk >> 16))
            plsc.store_scatter(wid, [qs], sv, mask=keep)
            return 0
        lax.fori_loop(0, VPC, vec_body, 0)
        return 0
    lax.fori_loop(0, NCH, chunk_body, 0)

    def process_window(w, jbuf, outT, osem, cnt_prev, warm):
        wq = w * KW

        @pl.when(warm)
        def _():
            pltpu.make_async_copy(
                outT, out.at[bt, :, pl.ds(sbase, KW)], osem).wait()
            ngp = (cnt_prev + 15) // 16

            def rz(g, _):
                jv = jbuf[pl.ds(g * 16, 16)]
                mask = (g * 16 + iota) < cnt_prev

                def rzc(c8, _):
                    for dc in range(8):
                        cs = jnp.broadcast_to(c8 * 8 + dc, (16,))
                        plsc.store_scatter(outT, [cs, jv], z16f, mask=mask)
                    return 0
                lax.fori_loop(0, C // 8, rzc, 0)
                return 0
            lax.fori_loop(0, ngp, rz, 0)

        def scan_chunk(k, cnt):
            wv = wid[pl.ds(wq + k * 16, 16)]
            mask = wv > 0
            plsc.store_compressed(jbuf.at[pl.ds(cnt, 16)], k * 16 + iota,
                                  mask=mask)
            plsc.store_compressed(idxbuf.at[pl.ds(cnt, 16)], wv - 1, mask=mask)
            return cnt + jnp.max(plsc.all_reduce_population_count(mask))
        cnt = lax.fori_loop(0, KW // 16, scan_chunk, jnp.int32(0))
        idxbuf[pl.ds(cnt, 16)] = z16i
        ng = (cnt + 15) // 16

        def gissue(g, _):
            idxv = idxbuf[pl.ds(g * 16, 16)]
            pltpu.async_copy(feat.at[idxv], rows.at[pl.ds(g * 16, 16)], gsem)
            return 0
        lax.fori_loop(0, ng, gissue, 0)

        def gdrain(g, _):
            pltpu.make_async_copy(
                feat.at[z16i], rows.at[pl.ds(g * 16, 16)], gsem).wait()
            return 0
        lax.fori_loop(0, ng, gdrain, 0)

        def sg(g, _):
            jv = jbuf[pl.ds(g * 16, 16)]
            uv = g * 16 + iota
            mask = uv < cnt

            def sgc(c8, _):
                for dc in range(8):
                    cs = jnp.broadcast_to(c8 * 8 + dc, (16,))
                    v = plsc.load_gather(rows, [uv, cs])
                    plsc.store_scatter(outT, [cs, jv], v, mask=mask)
                return 0
            lax.fori_loop(0, C // 8, sgc, 0)
            return 0
        lax.fori_loop(0, ng, sg, 0)

        pltpu.async_copy(outT, out.at[bt, :, pl.ds(sbase + wq, KW)], osem)
        return cnt

    def outer(i, carry):
        cntA, cntB = carry
        cntA = process_window(2 * i, jbufA, outTA, osemA, cntA, i >= 1)
        cntB = process_window(2 * i + 1, jbufB, outTB, osemB, cntB, i >= 1)
        return (cntA, cntB)
    lax.fori_loop(0, NWIN // 2, outer, (jnp.int32(0), jnp.int32(0)))

    pltpu.make_async_copy(outTA, out.at[bt, :, pl.ds(sbase, KW)], osemA).wait()
    pltpu.make_async_copy(outTB, out.at[bt, :, pl.ds(sbase, KW)], osemB).wait()


@jax.jit
def kernel(features, batch_idx, z_idx, y_idx, x_idx):
    mesh = plsc.VectorSubcoreMesh(core_axis_name="c", subcore_axis_name="s")
    run = pl.kernel(
        _body,
        out_type=jax.ShapeDtypeStruct((B, C, S), jnp.float32),
        mesh=mesh,
        compiler_params=pltpu.CompilerParams(
            use_tc_tiling_on_sc=False, needs_layout_passes=False),
        scratch_types=[
            pltpu.VMEM((TQ,), jnp.int32),
            pltpu.VMEM((SCAN,), jnp.int32),
            pltpu.VMEM((SCAN,), jnp.int32),
            pltpu.VMEM((SCAN,), jnp.int32),
            pltpu.VMEM((SCAN,), jnp.int32),
            pltpu.VMEM((32,), jnp.int32),
            pltpu.VMEM((GCAP,), jnp.int32),
            pltpu.VMEM((GCAP,), jnp.int32),
            pltpu.VMEM((GCAP,), jnp.int32),
            pltpu.VMEM((KW, C), jnp.float32),
            pltpu.VMEM((C, KW), jnp.float32),
            pltpu.VMEM((C, KW), jnp.float32),
            pltpu.SemaphoreType.DMA,
            pltpu.SemaphoreType.DMA,
            pltpu.SemaphoreType.DMA,
        ],
    )
    dense = run(features, batch_idx, z_idx, y_idx, x_idx)
    return dense.reshape(B, C, D, H, W)


# speedup vs baseline: 1.1099x; 1.0005x over previous
import functools

import jax
import jax.numpy as jnp
from jax import lax
from jax.experimental import pallas as pl
from jax.experimental.pallas import tpu as pltpu
from jax.experimental.pallas import tpu_sc as plsc

B, C, D, H, W = 4, 128, 2, 200, 176
S = D * H * W
Q = B * S
N = 40000
NT = 32
TQ = Q // NT
KW = 176
NWIN = TQ // KW
SCAN = 4000
NCH = N // SCAN
VPC = SCAN // 16
GCAP = 208
IMAX = 0x7FFFFFFF


def _body(feat, bidx, zidx, yidx, xidx, out,
          wid, bbuf, zbuf, ybuf, xbuf, sc17,
          jbufA, jbufB, idxbuf, rows, outTA, outTB,
          gsem, osemA, osemB):
    t = lax.axis_index("c") * 16 + lax.axis_index("s")
    q0 = t * TQ
    bt = t // 8
    sbase = (t % 8) * TQ
    iota = lax.iota(jnp.int32, 16)
    z16f = jnp.zeros((16,), jnp.float32)
    z16i = jnp.zeros((16,), jnp.int32)

    def zwid(k, _):
        wid[pl.ds(k * 16, 16)] = z16i
        return 0
    lax.fori_loop(0, TQ // 16, zwid, 0)
    sc17[pl.ds(16, 16)] = jnp.full((16,), -1, jnp.int32)

    def zot(k, _):
        c = k // (KW // 16)
        o = (k % (KW // 16)) * 16
        outTA[c, pl.ds(o, 16)] = z16f
        outTB[c, pl.ds(o, 16)] = z16f
        return 0
    lax.fori_loop(0, C * (KW // 16), zot, 0)

    def chunk_body(ch, _):
        off = ch * SCAN
        pltpu.sync_copy(bidx.at[pl.ds(off, SCAN)], bbuf)
        pltpu.sync_copy(zidx.at[pl.ds(off, SCAN)], zbuf)
        pltpu.sync_copy(yidx.at[pl.ds(off, SCAN)], ybuf)
        pltpu.sync_copy(xidx.at[pl.ds(off, SCAN)], xbuf)

        def vec_body(k, _):
            n_vec = (off + k * 16) + iota
            bv = bbuf[pl.ds(k * 16, 16)]
            zv = zbuf[pl.ds(k * 16, 16)]
            yv = ybuf[pl.ds(k * 16, 16)]
            xv = xbuf[pl.ds(k * 16, 16)]
            ql = ((bv * D + zv) * H + yv) * W + xv - q0
            inr = (ql >= 0) & (ql < TQ)
            key = jnp.where(inr, (ql << 16) | n_vec, IMAX)
            sk, sv = plsc.sort_key_val(key, n_vec + 1)
            qs = sk >> 16
            sc17[pl.ds(0, 16)] = sk
            nk = plsc.load_gather(sc17, [iota + 1])
            keep = (qs < TQ) & (qs != (nk >> 16))
            plsc.store_scatter(wid, [qs], sv, mask=keep)
            return 0
        lax.fori_loop(0, VPC, vec_body, 0)
        return 0
    with jax.named_scope("p1_scan"):
        lax.fori_loop(0, NCH, chunk_body, 0)

    def process_window(w, jbuf, outT, osem, cnt_prev, warm):
        wq = w * KW

        @pl.when(warm)
        def _():
            pltpu.make_async_copy(
                outT, out.at[bt, :, pl.ds(sbase, KW)], osem).wait()
            ngp = (cnt_prev + 15) // 16

            def rz(g, _):
                jv = jbuf[pl.ds(g * 16, 16)]
                mask = (g * 16 + iota) < cnt_prev

                def rzc(c8, _):
                    for dc in range(8):
                        cs = jnp.broadcast_to(c8 * 8 + dc, (16,))
                        plsc.store_scatter(outT, [cs, jv], z16f, mask=mask)
                    return 0
                lax.fori_loop(0, C // 8, rzc, 0)
                return 0
            lax.fori_loop(0, ngp, rz, 0)

        def scan_chunk(k, cnt):
            wv = wid[pl.ds(wq + k * 16, 16)]
            mask = wv > 0
            plsc.store_compressed(jbuf.at[pl.ds(cnt, 16)], k * 16 + iota,
                                  mask=mask)
            plsc.store_compressed(idxbuf.at[pl.ds(cnt, 16)], wv - 1, mask=mask)
            return cnt + jnp.max(plsc.all_reduce_population_count(mask))
        cnt = lax.fori_loop(0, KW // 16, scan_chunk, jnp.int32(0))
        idxbuf[pl.ds(cnt, 16)] = z16i
        ng = (cnt + 15) // 16

        def gissue(g, _):
            idxv = idxbuf[pl.ds(g * 16, 16)]
            pltpu.async_copy(feat.at[idxv], rows.at[pl.ds(g * 16, 16)], gsem)
            return 0
        lax.fori_loop(0, ng, gissue, 0)

        def gdrain(g, _):
            pltpu.make_async_copy(
                feat.at[z16i], rows.at[pl.ds(g * 16, 16)], gsem).wait()
            return 0
        lax.fori_loop(0, ng, gdrain, 0)

        def sg(g, _):
            jv = jbuf[pl.ds(g * 16, 16)]
            uv = g * 16 + iota
            mask = uv < cnt

            def sgc(c8, _):
                for dc in range(8):
                    cs = jnp.broadcast_to(c8 * 8 + dc, (16,))
                    v = plsc.load_gather(rows, [uv, cs])
                    plsc.store_scatter(outT, [cs, jv], v, mask=mask)
                return 0
            lax.fori_loop(0, C // 8, sgc, 0)
            return 0
        lax.fori_loop(0, ng, sg, 0)

        pltpu.async_copy(outT, out.at[bt, :, pl.ds(sbase + wq, KW)], osem)
        return cnt

    def outer(i, carry):
        cntA, cntB = carry
        cntA = process_window(2 * i, jbufA, outTA, osemA, cntA, i >= 1)
        cntB = process_window(2 * i + 1, jbufB, outTB, osemB, cntB, i >= 1)
        return (cntA, cntB)
    with jax.named_scope("p2_windows"):
        lax.fori_loop(0, NWIN // 2, outer, (jnp.int32(0), jnp.int32(0)))

    pltpu.make_async_copy(outTA, out.at[bt, :, pl.ds(sbase, KW)], osemA).wait()
    pltpu.make_async_copy(outTB, out.at[bt, :, pl.ds(sbase, KW)], osemB).wait()


@jax.jit
def kernel(features, batch_idx, z_idx, y_idx, x_idx):
    mesh = plsc.VectorSubcoreMesh(core_axis_name="c", subcore_axis_name="s")
    run = pl.kernel(
        _body,
        out_type=jax.ShapeDtypeStruct((B, C, S), jnp.float32),
        mesh=mesh,
        compiler_params=pltpu.CompilerParams(
            use_tc_tiling_on_sc=False, needs_layout_passes=False),
        scratch_types=[
            pltpu.VMEM((TQ,), jnp.int32),
            pltpu.VMEM((SCAN,), jnp.int32),
            pltpu.VMEM((SCAN,), jnp.int32),
            pltpu.VMEM((SCAN,), jnp.int32),
            pltpu.VMEM((SCAN,), jnp.int32),
            pltpu.VMEM((32,), jnp.int32),
            pltpu.VMEM((GCAP,), jnp.int32),
            pltpu.VMEM((GCAP,), jnp.int32),
            pltpu.VMEM((GCAP,), jnp.int32),
            pltpu.VMEM((KW, C), jnp.float32),
            pltpu.VMEM((C, KW), jnp.float32),
            pltpu.VMEM((C, KW), jnp.float32),
            pltpu.SemaphoreType.DMA,
            pltpu.SemaphoreType.DMA,
            pltpu.SemaphoreType.DMA,
        ],
    )
    dense = run(features, batch_idx, z_idx, y_idx, x_idx)
    return dense.reshape(B, C, D, H, W)


# speedup vs baseline: 1.1104x; 1.0004x over previous
import functools

import jax
import jax.numpy as jnp
from jax import lax
from jax.experimental import pallas as pl
from jax.experimental.pallas import tpu as pltpu
from jax.experimental.pallas import tpu_sc as plsc

B, C, D, H, W = 4, 128, 2, 200, 176
S = D * H * W
Q = B * S
N = 40000
NT = 32
TQ = Q // NT
KW = 176
NWIN = TQ // KW
SCAN = 4000
NCH = N // SCAN
VPC = SCAN // 16
GCAP = 208
IMAX = 0x7FFFFFFF


def _body(feat, bidx, zidx, yidx, xidx, out,
          wid, bbuf, zbuf, ybuf, xbuf, sc17,
          jbufA, jbufB, idxbuf, rows, outTA, outTB,
          gsem, osemA, osemB):
    t = lax.axis_index("c") * 16 + lax.axis_index("s")
    q0 = t * TQ
    bt = t // 8
    sbase = (t % 8) * TQ
    iota = lax.iota(jnp.int32, 16)
    z16f = jnp.zeros((16,), jnp.float32)
    z16i = jnp.zeros((16,), jnp.int32)

    def zwid(k, _):
        wid[pl.ds(k * 16, 16)] = z16i
        return 0
    lax.fori_loop(0, TQ // 16, zwid, 0)
    sc17[pl.ds(16, 16)] = jnp.full((16,), -1, jnp.int32)

    def zot(k, _):
        c = k // (KW // 16)
        o = (k % (KW // 16)) * 16
        outTA[c, pl.ds(o, 16)] = z16f
        outTB[c, pl.ds(o, 16)] = z16f
        return 0
    lax.fori_loop(0, C * (KW // 16), zot, 0)

    def chunk_body(ch, _):
        off = ch * SCAN
        pltpu.sync_copy(bidx.at[pl.ds(off, SCAN)], bbuf)
        pltpu.sync_copy(zidx.at[pl.ds(off, SCAN)], zbuf)
        pltpu.sync_copy(yidx.at[pl.ds(off, SCAN)], ybuf)
        pltpu.sync_copy(xidx.at[pl.ds(off, SCAN)], xbuf)

        def vec_body(k, _):
            n_vec = (off + k * 16) + iota
            bv = bbuf[pl.ds(k * 16, 16)]
            zv = zbuf[pl.ds(k * 16, 16)]
            yv = ybuf[pl.ds(k * 16, 16)]
            xv = xbuf[pl.ds(k * 16, 16)]
            ql = ((bv * D + zv) * H + yv) * W + xv - q0
            inr = (ql >= 0) & (ql < TQ)
            key = jnp.where(inr, (ql << 16) | n_vec, IMAX)
            sk, sv = plsc.sort_key_val(key, n_vec + 1)
            qs = sk >> 16
            sc17[pl.ds(0, 16)] = sk
            nk = plsc.load_gather(sc17, [iota + 1])
            keep = (qs < TQ) & (qs != (nk >> 16))
            plsc.store_scatter(wid, [qs], sv, mask=keep)
            return 0
        lax.fori_loop(0, VPC, vec_body, 0)
        return 0
    with jax.named_scope("p1_scan"):
        lax.fori_loop(0, NCH, chunk_body, 0)

    def process_window(w, jbuf, outT, osem, cnt_prev, warm):
        wq = w * KW

        @pl.when(warm)
        def _():
            with jax.named_scope("w_drain_out"):
                pltpu.make_async_copy(
                    outT, out.at[bt, :, pl.ds(sbase, KW)], osem).wait()
            ngp = (cnt_prev + 15) // 16

            def rz(g, _):
                jv = jbuf[pl.ds(g * 16, 16)]
                mask = (g * 16 + iota) < cnt_prev

                def rzc(c8, _):
                    for dc in range(8):
                        cs = jnp.broadcast_to(c8 * 8 + dc, (16,))
                        plsc.store_scatter(outT, [cs, jv], z16f, mask=mask)
                    return 0
                lax.fori_loop(0, C // 8, rzc, 0)
                return 0
            lax.fori_loop(0, ngp, rz, 0)

        def scan_chunk(k, cnt):
            wv = wid[pl.ds(wq + k * 16, 16)]
            mask = wv > 0
            plsc.store_compressed(jbuf.at[pl.ds(cnt, 16)], k * 16 + iota,
                                  mask=mask)
            plsc.store_compressed(idxbuf.at[pl.ds(cnt, 16)], wv - 1, mask=mask)
            return cnt + jnp.max(plsc.all_reduce_population_count(mask))
        with jax.named_scope("w_scan"):
            cnt = lax.fori_loop(0, KW // 16, scan_chunk, jnp.int32(0))
        idxbuf[pl.ds(cnt, 16)] = z16i
        ng = (cnt + 15) // 16

        def gissue(g, _):
            idxv = idxbuf[pl.ds(g * 16, 16)]
            pltpu.async_copy(feat.at[idxv], rows.at[pl.ds(g * 16, 16)], gsem)
            return 0
        lax.fori_loop(0, ng, gissue, 0)

        def gdrain(g, _):
            pltpu.make_async_copy(
                feat.at[z16i], rows.at[pl.ds(g * 16, 16)], gsem).wait()
            return 0
        with jax.named_scope("w_gdrain"):
            lax.fori_loop(0, ng, gdrain, 0)

        def sg(g, _):
            jv = jbuf[pl.ds(g * 16, 16)]
            uv = g * 16 + iota
            mask = uv < cnt

            def sgc(c8, _):
                for dc in range(8):
                    cs = jnp.broadcast_to(c8 * 8 + dc, (16,))
                    v = plsc.load_gather(rows, [uv, cs])
                    plsc.store_scatter(outT, [cs, jv], v, mask=mask)
                return 0
            lax.fori_loop(0, C // 8, sgc, 0)
            return 0
        with jax.named_scope("w_scatter"):
            lax.fori_loop(0, ng, sg, 0)

        pltpu.async_copy(outT, out.at[bt, :, pl.ds(sbase + wq, KW)], osem)
        return cnt

    def outer(i, carry):
        cntA, cntB = carry
        cntA = process_window(2 * i, jbufA, outTA, osemA, cntA, i >= 1)
        cntB = process_window(2 * i + 1, jbufB, outTB, osemB, cntB, i >= 1)
        return (cntA, cntB)
    with jax.named_scope("p2_windows"):
        lax.fori_loop(0, NWIN // 2, outer, (jnp.int32(0), jnp.int32(0)))

    pltpu.make_async_copy(outTA, out.at[bt, :, pl.ds(sbase, KW)], osemA).wait()
    pltpu.make_async_copy(outTB, out.at[bt, :, pl.ds(sbase, KW)], osemB).wait()


@jax.jit
def kernel(features, batch_idx, z_idx, y_idx, x_idx):
    mesh = plsc.VectorSubcoreMesh(core_axis_name="c", subcore_axis_name="s")
    run = pl.kernel(
        _body,
        out_type=jax.ShapeDtypeStruct((B, C, S), jnp.float32),
        mesh=mesh,
        compiler_params=pltpu.CompilerParams(
            use_tc_tiling_on_sc=False, needs_layout_passes=False),
        scratch_types=[
            pltpu.VMEM((TQ,), jnp.int32),
            pltpu.VMEM((SCAN,), jnp.int32),
            pltpu.VMEM((SCAN,), jnp.int32),
            pltpu.VMEM((SCAN,), jnp.int32),
            pltpu.VMEM((SCAN,), jnp.int32),
            pltpu.VMEM((32,), jnp.int32),
            pltpu.VMEM((GCAP,), jnp.int32),
            pltpu.VMEM((GCAP,), jnp.int32),
            pltpu.VMEM((GCAP,), jnp.int32),
            pltpu.VMEM((KW, C), jnp.float32),
            pltpu.VMEM((C, KW), jnp.float32),
            pltpu.VMEM((C, KW), jnp.float32),
            pltpu.SemaphoreType.DMA,
            pltpu.SemaphoreType.DMA,
            pltpu.SemaphoreType.DMA,
        ],
    )
    dense = run(features, batch_idx, z_idx, y_idx, x_idx)
    return dense.reshape(B, C, D, H, W)


# speedup vs baseline: 2.2034x; 1.9843x over previous
import jax
import jax.numpy as jnp
from jax import lax
from jax.experimental import pallas as pl
from jax.experimental.pallas import tpu as pltpu
from jax.experimental.pallas import tpu_sc as plsc

B, C, D, H, W = 4, 128, 2, 200, 176
S = D * H * W
Q = B * S
N = 40000
NT = 32
KW = 128
NWG = Q // KW
WPB = S // KW
NWJ = (NWG + NT - 1) // NT
TQL = NWJ * KW
GCAP = TQL + KW
RING = 512
SCAN = 1024
NCH = 39
TAIL = N - NCH * SCAN
IMAX = 0x7FFFFFFF


def _body(feat, bidx, zidx, yidx, xidx, out,
          wid, bbuf, zbuf, ybuf, xbuf, sc64,
          jlist, idxlist, ring, outTA, outTB, starts,
          gsem, osemA, osemB):
    t = lax.axis_index("c") * 16 + lax.axis_index("s")
    nw_t = jnp.where(t < NWG - (NWJ - 1) * NT, NWJ, NWJ - 1)
    iota = lax.iota(jnp.int32, 16)
    z16f = jnp.zeros((16,), jnp.float32)
    z16i = jnp.zeros((16,), jnp.int32)
    cvecs = [c8 * 16 + iota for c8 in range(8)]

    def zwid(k, _):
        wid[pl.ds(k * 16, 16)] = z16i
        return 0
    lax.fori_loop(0, TQL // 16, zwid, 0)
    sc64[pl.ds(16, 16)] = jnp.full((16,), -1, jnp.int32)
    sc64[pl.ds(48, 16)] = jnp.full((16,), -1, jnp.int32)

    def zot(k, _):
        c = k // (KW // 16)
        o = (k % (KW // 16)) * 16
        outTA[c, pl.ds(o, 16)] = z16f
        outTB[c, pl.ds(o, 16)] = z16f
        return 0
    lax.fori_loop(0, C * (KW // 16), zot, 0)

    def win_key(k2, half, n_vec):
        base = k2 * 32 + half * 16
        bv = bbuf[pl.ds(base, 16)]
        zv = zbuf[pl.ds(base, 16)]
        yv = ybuf[pl.ds(base, 16)]
        xv = xbuf[pl.ds(base, 16)]
        qv = ((bv * D + zv) * H + yv) * W + xv
        wk = qv >> 7
        inr = (wk & (NT - 1)) == t
        jloc = ((wk >> 5) << 7) | (qv & (KW - 1))
        return jnp.where(inr, (jloc << 16) | n_vec, IMAX)

    def scan_pair(off, k2):
        n0 = (off + k2 * 32) + iota
        n1 = n0 + 16
        key0 = win_key(k2, 0, n0)
        key1 = win_key(k2, 1, n1)
        sk0, sv0 = plsc.sort_key_val(key0, n0 + 1)
        sk1, sv1 = plsc.sort_key_val(key1, n1 + 1)
        sc64[pl.ds(0, 16)] = sk0
        sc64[pl.ds(32, 16)] = sk1
        nk0 = plsc.load_gather(sc64, [iota + 1])
        nk1 = plsc.load_gather(sc64, [iota + 33])
        q0 = sk0 >> 16
        q1 = sk1 >> 16
        keep0 = (q0 < TQL) & (q0 != (nk0 >> 16))
        keep1 = (q1 < TQL) & (q1 != (nk1 >> 16))
        plsc.store_scatter(wid, [q0], sv0, mask=keep0)
        plsc.store_scatter(wid, [q1], sv1, mask=keep1)

    def chunk_body(ch, _):
        off = pl.multiple_of(ch * SCAN, SCAN)
        pltpu.sync_copy(bidx.at[pl.ds(off, SCAN)], bbuf)
        pltpu.sync_copy(zidx.at[pl.ds(off, SCAN)], zbuf)
        pltpu.sync_copy(yidx.at[pl.ds(off, SCAN)], ybuf)
        pltpu.sync_copy(xidx.at[pl.ds(off, SCAN)], xbuf)

        def vec_body(k2, _):
            scan_pair(off, k2)
            return 0
        lax.fori_loop(0, SCAN // 32, vec_body, 0)
        return 0

    with jax.named_scope("p1_scan"):
        lax.fori_loop(0, NCH, chunk_body, 0)
        toff = NCH * SCAN
        pltpu.sync_copy(bidx.at[pl.ds(toff, TAIL)], bbuf.at[pl.ds(0, TAIL)])
        pltpu.sync_copy(zidx.at[pl.ds(toff, TAIL)], zbuf.at[pl.ds(0, TAIL)])
        pltpu.sync_copy(yidx.at[pl.ds(toff, TAIL)], ybuf.at[pl.ds(0, TAIL)])
        pltpu.sync_copy(xidx.at[pl.ds(toff, TAIL)], xbuf.at[pl.ds(0, TAIL)])

        def tail_body(k2, _):
            scan_pair(toff, k2)
            return 0
        lax.fori_loop(0, TAIL // 32, tail_body, 0)

    with jax.named_scope("p2a_compact"):
        starts[0] = jnp.int32(0)

        def scanw(lw, cnt):
            def sck(k, c):
                wv = wid[pl.ds(lw * KW + k * 16, 16)]
                m = wv > 0
                plsc.store_compressed(jlist.at[pl.ds(c, 16)], k * 16 + iota,
                                      mask=m)
                plsc.store_compressed(idxlist.at[pl.ds(c, 16)], wv - 1,
                                      mask=m)
                return c + jnp.max(plsc.all_reduce_population_count(m))
            cnt = lax.fori_loop(0, KW // 16, sck, cnt)
            starts[lw + 1] = cnt
            return cnt
        U = lax.fori_loop(0, nw_t, scanw, jnp.int32(0))

        def phantom(lw, _):
            starts[lw + 1] = U
            return 0
        lax.fori_loop(nw_t, NWJ + 1, phantom, 0)

        def padi(k, _):
            idxlist[pl.ds(U + k * 16, 16)] = z16i
            return 0
        lax.fori_loop(0, KW // 16, padi, 0)
        nd = (U + KW - 1) // KW

    def process_window(lw, outT, osem, dI, dR):
        live = lw < nw_t
        start_w = starts[jnp.minimum(lw, NWJ)]
        end_w = starts[jnp.minimum(lw, NWJ) + 1]

        @pl.when((lw >= 2) & live)
        def _():
            pltpu.make_async_copy(
                outT, out.at[0, :, pl.ds(0, KW)], osem).wait()
            s_p = starts[lw - 2]
            e_p = starts[lw - 1]

            def rz(u, _):
                col = jlist[pl.ds(u, 16)][0]
                bc = jnp.broadcast_to(col, (16,))
                for c8 in range(8):
                    plsc.store_scatter(outT, [cvecs[c8], bc], z16f)
                return 0
            lax.fori_loop(s_p, e_p, rz, 0)

        def icond(d):
            return ((d < nd) & (d * KW < end_w + 2 * KW)
                    & ((d < 4) | ((d - 3) * KW <= start_w)))

        def ibody(d):
            slot = (d & 3) * KW
            pltpu.async_copy(feat.at[idxlist.at[pl.ds(d * KW, KW)]],
                             ring.at[pl.ds(slot, KW)], gsem)
            return d + 1
        dI = lax.while_loop(icond, ibody, dI)

        need = (end_w + KW - 1) // KW

        def dbody(d):
            pltpu.make_async_copy(feat.at[idxlist.at[pl.ds(0, KW)]],
                                  ring.at[pl.ds(0, KW)], gsem).wait()
            return d + 1
        dR = lax.while_loop(lambda d: d < need, dbody, dR)

        def sg(u, _):
            col = jlist[pl.ds(u, 16)][0]
            bc = jnp.broadcast_to(col, (16,))
            r = u & (RING - 1)
            for c8 in range(8):
                v = ring[r, pl.ds(c8 * 16, 16)]
                plsc.store_scatter(outT, [cvecs[c8], bc], v)
            return 0
        lax.fori_loop(start_w, end_w, sg, 0)

        @pl.when(live)
        def _():
            gw = t + NT * lw
            b = gw // WPB
            s0 = pl.multiple_of((gw % WPB) * KW, KW)
            pltpu.async_copy(outT, out.at[b, :, pl.ds(s0, KW)], osem)
        return dI, dR

    with jax.named_scope("p2b_windows"):
        def outer(i, carry):
            dI, dR = carry
            dI, dR = process_window(2 * i, outTA, osemA, dI, dR)
            dI, dR = process_window(2 * i + 1, outTB, osemB, dI, dR)
            return (dI, dR)
        lax.fori_loop(0, (NWJ + 1) // 2, outer,
                      (jnp.int32(0), jnp.int32(0)))

    pltpu.make_async_copy(outTA, out.at[0, :, pl.ds(0, KW)], osemA).wait()
    pltpu.make_async_copy(outTB, out.at[0, :, pl.ds(0, KW)], osemB).wait()


@jax.jit
def kernel(features, batch_idx, z_idx, y_idx, x_idx):
    mesh = plsc.VectorSubcoreMesh(core_axis_name="c", subcore_axis_name="s")
    run = pl.kernel(
        _body,
        out_type=jax.ShapeDtypeStruct((B, C, S), jnp.float32),
        mesh=mesh,
        compiler_params=pltpu.CompilerParams(
            use_tc_tiling_on_sc=True, needs_layout_passes=False),
        scratch_types=[
            pltpu.VMEM((TQL,), jnp.int32),
            pltpu.VMEM((SCAN,), jnp.int32),
            pltpu.VMEM((SCAN,), jnp.int32),
            pltpu.VMEM((SCAN,), jnp.int32),
            pltpu.VMEM((SCAN,), jnp.int32),
            pltpu.VMEM((64,), jnp.int32),
            pltpu.VMEM((GCAP,), jnp.int32),
            pltpu.VMEM((GCAP,), jnp.int32),
            pltpu.VMEM((RING, C), jnp.float32),
            pltpu.VMEM((C, KW), jnp.float32),
            pltpu.VMEM((C, KW), jnp.float32),
            pltpu.SMEM((NWJ + 2,), jnp.int32),
            pltpu.SemaphoreType.DMA,
            pltpu.SemaphoreType.DMA,
            pltpu.SemaphoreType.DMA,
        ],
    )
    dense = run(features, batch_idx, z_idx, y_idx, x_idx)
    return dense.reshape(B, C, D, H, W)
